# baseline (speedup 1.0000x reference)
import functools

import jax
import jax.numpy as jnp
from jax.experimental import pallas as pl
from jax.experimental.pallas import tpu as pltpu
from jax.experimental.pallas import tpu_sc as plsc

_GATHER_WINDOW = 128
_MLP_BLOCK = 2048


def _sc_gather(table, idx):
    B = idx.shape[0]
    D = table.shape[1]
    idx2 = idx.reshape(1, B)
    mesh = plsc.VectorSubcoreMesh(core_axis_name="c", subcore_axis_name="s")

    @functools.partial(
        pl.kernel,
        out_type=jax.ShapeDtypeStruct((B, D), table.dtype),
        mesh=mesh,
    )
    def gather_kernel(table_hbm, i_hbm, o_hbm):
        def body(i_vmem, o_vmem):
            pltpu.sync_copy(table_hbm.at[i_vmem.at[0]], o_vmem)

        pltpu.emit_pipeline(
            body,
            grid=(B // _GATHER_WINDOW,),
            in_specs=[pl.BlockSpec((1, _GATHER_WINDOW), lambda i: (0, i))],
            out_specs=[pl.BlockSpec((_GATHER_WINDOW, D), lambda i: (i, 0))],
            core_axis_name=("c", "s"),
            dimension_semantics=(pltpu.PARALLEL,),
        )(i_hbm, o_hbm)

    return gather_kernel(table, idx2)


def _mlp_body(h_ref, w1_ref, b1_ref, w2_ref, b2_ref, w3_ref, b3_ref, o_ref):
    h = h_ref[...]
    z = jnp.dot(h, w1_ref[...], preferred_element_type=jnp.float32) + b1_ref[...]
    z = jnp.maximum(z, 0.0)
    z = jnp.dot(z, w2_ref[...], preferred_element_type=jnp.float32) + b2_ref[...]
    z = jnp.maximum(z, 0.0)
    z = jnp.dot(z, w3_ref[...], preferred_element_type=jnp.float32) + b3_ref[...]
    o_ref[...] = jax.nn.sigmoid(z)


def _mlp(h, W1, b1, W2, b2, W3, b3):
    B, D = h.shape
    grid = (B // _MLP_BLOCK,)
    return pl.pallas_call(
        _mlp_body,
        grid=grid,
        in_specs=[
            pl.BlockSpec((_MLP_BLOCK, D), lambda i: (i, 0)),
            pl.BlockSpec(W1.shape, lambda i: (0, 0)),
            pl.BlockSpec((1, b1.shape[0]), lambda i: (0, 0)),
            pl.BlockSpec(W2.shape, lambda i: (0, 0)),
            pl.BlockSpec((1, b2.shape[0]), lambda i: (0, 0)),
            pl.BlockSpec(W3.shape, lambda i: (0, 0)),
            pl.BlockSpec((1, b3.shape[0]), lambda i: (0, 0)),
        ],
        out_specs=pl.BlockSpec((_MLP_BLOCK, 1), lambda i: (i, 0)),
        out_shape=jax.ShapeDtypeStruct((B, 1), jnp.float32),
    )(h, W1, b1.reshape(1, -1), W2, b2.reshape(1, -1), W3, b3.reshape(1, -1))


_N_CHUNKS = 4


def kernel(x, table, W1, b1, W2, b2, W3, b3):
    x = x.astype(jnp.int32)
    B = x.shape[0]
    C = B // _N_CHUNKS
    outs = []
    for i in range(_N_CHUNKS):
        xi = jax.lax.slice(x, (i * C,), ((i + 1) * C,))
        hi = _sc_gather(table, xi)
        outs.append(_mlp(hi, W1, b1, W2, b2, W3, b3))
    return jnp.concatenate(outs, axis=0)


# speedup vs baseline: 1.0959x; 1.0959x over previous
import functools

import jax
import jax.numpy as jnp
from jax.experimental import pallas as pl
from jax.experimental.pallas import tpu as pltpu
from jax.experimental.pallas import tpu_sc as plsc

_GATHER_WINDOW = 128
_MLP_BLOCK = 2048


def _sc_gather(table, idx):
    B = idx.shape[0]
    D = table.shape[1]
    idx2 = idx.reshape(1, B)
    mesh = plsc.VectorSubcoreMesh(core_axis_name="c", subcore_axis_name="s")

    @functools.partial(
        pl.kernel,
        out_type=jax.ShapeDtypeStruct((B, D), table.dtype),
        mesh=mesh,
    )
    def gather_kernel(table_hbm, i_hbm, o_hbm):
        def body(i_vmem, o_vmem):
            pltpu.sync_copy(table_hbm.at[i_vmem.at[0]], o_vmem)

        pltpu.emit_pipeline(
            body,
            grid=(B // _GATHER_WINDOW,),
            in_specs=[pl.BlockSpec((1, _GATHER_WINDOW), lambda i: (0, i))],
            out_specs=[pl.BlockSpec((_GATHER_WINDOW, D), lambda i: (i, 0))],
            core_axis_name=("c", "s"),
            dimension_semantics=(pltpu.PARALLEL,),
        )(i_hbm, o_hbm)

    return gather_kernel(table, idx2)


def _mlp_body(h_ref, w1_ref, b1_ref, w2_ref, b2_ref, w3_ref, b3_ref, o_ref):
    h = h_ref[...].astype(jnp.bfloat16)
    z = jnp.dot(h, w1_ref[...].astype(jnp.bfloat16),
                preferred_element_type=jnp.float32) + b1_ref[...]
    z = jnp.maximum(z, 0.0).astype(jnp.bfloat16)
    z = jnp.dot(z, w2_ref[...].astype(jnp.bfloat16),
                preferred_element_type=jnp.float32) + b2_ref[...]
    z = jnp.maximum(z, 0.0).astype(jnp.bfloat16)
    z = jnp.dot(z, w3_ref[...].astype(jnp.bfloat16),
                preferred_element_type=jnp.float32) + b3_ref[...]
    o_ref[...] = jax.nn.sigmoid(z)


def _mlp(h, W1, b1, W2, b2, W3, b3):
    B, D = h.shape
    grid = (B // _MLP_BLOCK,)
    return pl.pallas_call(
        _mlp_body,
        grid=grid,
        in_specs=[
            pl.BlockSpec((_MLP_BLOCK, D), lambda i: (i, 0)),
            pl.BlockSpec(W1.shape, lambda i: (0, 0)),
            pl.BlockSpec((1, b1.shape[0]), lambda i: (0, 0)),
            pl.BlockSpec(W2.shape, lambda i: (0, 0)),
            pl.BlockSpec((1, b2.shape[0]), lambda i: (0, 0)),
            pl.BlockSpec(W3.shape, lambda i: (0, 0)),
            pl.BlockSpec((1, b3.shape[0]), lambda i: (0, 0)),
        ],
        out_specs=pl.BlockSpec((_MLP_BLOCK, 1), lambda i: (i, 0)),
        out_shape=jax.ShapeDtypeStruct((B, 1), jnp.float32),
    )(h, W1, b1.reshape(1, -1), W2, b2.reshape(1, -1), W3, b3.reshape(1, -1))


def kernel(x, table, W1, b1, W2, b2, W3, b3):
    x = x.astype(jnp.int32)
    h = _sc_gather(table, x)
    return _mlp(h, W1, b1, W2, b2, W3, b3)


# speedup vs baseline: 1.1821x; 1.0786x over previous
import functools

import jax
import jax.numpy as jnp
from jax.experimental import pallas as pl
from jax.experimental.pallas import tpu as pltpu
from jax.experimental.pallas import tpu_sc as plsc

_GATHER_WINDOW = 128
_MLP_BLOCK = 8192


def _sc_gather(table, idx):
    B = idx.shape[0]
    D = table.shape[1]
    idx2 = idx.reshape(1, B)
    mesh = plsc.VectorSubcoreMesh(core_axis_name="c", subcore_axis_name="s")

    @functools.partial(
        pl.kernel,
        out_type=jax.ShapeDtypeStruct((B, D), table.dtype),
        mesh=mesh,
    )
    def gather_kernel(table_hbm, i_hbm, o_hbm):
        def body(i_vmem, o_vmem):
            pltpu.sync_copy(table_hbm.at[i_vmem.at[0]], o_vmem)

        pltpu.emit_pipeline(
            body,
            grid=(B // _GATHER_WINDOW,),
            in_specs=[pl.BlockSpec((1, _GATHER_WINDOW), lambda i: (0, i))],
            out_specs=[pl.BlockSpec((_GATHER_WINDOW, D), lambda i: (i, 0))],
            core_axis_name=("c", "s"),
            dimension_semantics=(pltpu.PARALLEL,),
        )(i_hbm, o_hbm)

    return gather_kernel(table, idx2)


def _mlp_body(h_ref, w1_ref, b1_ref, w2_ref, b2_ref, w3_ref, b3_ref, o_ref):
    h = h_ref[...].astype(jnp.bfloat16)
    z = jnp.dot(h, w1_ref[...].astype(jnp.bfloat16),
                preferred_element_type=jnp.float32) + b1_ref[...]
    z = jnp.maximum(z, 0.0).astype(jnp.bfloat16)
    z = jnp.dot(z, w2_ref[...].astype(jnp.bfloat16),
                preferred_element_type=jnp.float32) + b2_ref[...]
    z = jnp.maximum(z, 0.0).astype(jnp.bfloat16)
    z = jnp.dot(z, w3_ref[...].astype(jnp.bfloat16),
                preferred_element_type=jnp.float32) + b3_ref[...]
    o_ref[...] = jax.nn.sigmoid(z)


def _mlp(h, W1, b1, W2, b2, W3, b3):
    B, D = h.shape
    grid = (B // _MLP_BLOCK,)
    return pl.pallas_call(
        _mlp_body,
        grid=grid,
        in_specs=[
            pl.BlockSpec((_MLP_BLOCK, D), lambda i: (i, 0)),
            pl.BlockSpec(W1.shape, lambda i: (0, 0)),
            pl.BlockSpec((1, b1.shape[0]), lambda i: (0, 0)),
            pl.BlockSpec(W2.shape, lambda i: (0, 0)),
            pl.BlockSpec((1, b2.shape[0]), lambda i: (0, 0)),
            pl.BlockSpec(W3.shape, lambda i: (0, 0)),
            pl.BlockSpec((1, b3.shape[0]), lambda i: (0, 0)),
        ],
        out_specs=pl.BlockSpec((_MLP_BLOCK, 1), lambda i: (i, 0)),
        out_shape=jax.ShapeDtypeStruct((B, 1), jnp.float32),
    )(h, W1, b1.reshape(1, -1), W2, b2.reshape(1, -1), W3, b3.reshape(1, -1))


def kernel(x, table, W1, b1, W2, b2, W3, b3):
    x = x.astype(jnp.int32)
    h = _sc_gather(table, x)
    return _mlp(h, W1, b1, W2, b2, W3, b3)


# speedup vs baseline: 1.5083x; 1.2760x over previous
import functools

import jax
import jax.numpy as jnp
from jax.experimental import pallas as pl
from jax.experimental.pallas import tpu as pltpu
from jax.experimental.pallas import tpu_sc as plsc

_GATHER_WINDOW = 128
_MLP_BLOCK = 8192


_NUM_UNITS = 32


def _sc_gather(table, idx):
    B = idx.shape[0]
    D = table.shape[1]
    per_unit = B // _NUM_UNITS
    n_win = per_unit // _GATHER_WINDOW
    mesh = plsc.VectorSubcoreMesh(core_axis_name="c", subcore_axis_name="s")

    row_scratch = [
        pltpu.VMEM((_GATHER_WINDOW, D), table.dtype) for _ in range(n_win)
    ]

    @functools.partial(
        pl.kernel,
        out_type=jax.ShapeDtypeStruct((B, D), table.dtype),
        mesh=mesh,
        scratch_types=[pltpu.VMEM((per_unit,), jnp.int32)]
        + row_scratch
        + [pltpu.SemaphoreType.DMA, pltpu.SemaphoreType.DMA],
    )
    def gather_kernel(table_hbm, i_hbm, o_hbm, idx_v, *rest):
        rows = rest[:n_win]
        sem_g, sem_o = rest[n_win], rest[n_win + 1]
        wid = jax.lax.axis_index("s") * 2 + jax.lax.axis_index("c")
        base = wid * per_unit
        pltpu.sync_copy(i_hbm.at[pl.ds(base, per_unit)], idx_v)
        gathers = [
            pltpu.async_copy(
                table_hbm.at[idx_v.at[pl.ds(k * _GATHER_WINDOW, _GATHER_WINDOW)]],
                rows[k],
                sem_g,
            )
            for k in range(n_win)
        ]
        writes = []
        for k in range(n_win):
            gathers[k].wait()
            writes.append(
                pltpu.async_copy(
                    rows[k],
                    o_hbm.at[pl.ds(base + k * _GATHER_WINDOW, _GATHER_WINDOW)],
                    sem_o,
                )
            )
        for w in writes:
            w.wait()

    return gather_kernel(table, idx)


def _mlp_body(h_ref, w1_ref, b1_ref, w2_ref, b2_ref, w3_ref, b3_ref, o_ref):
    h = h_ref[...].astype(jnp.bfloat16)
    z = jnp.dot(h, w1_ref[...].astype(jnp.bfloat16),
                preferred_element_type=jnp.float32) + b1_ref[...]
    z = jnp.maximum(z, 0.0).astype(jnp.bfloat16)
    z = jnp.dot(z, w2_ref[...].astype(jnp.bfloat16),
                preferred_element_type=jnp.float32) + b2_ref[...]
    z = jnp.maximum(z, 0.0)
    z = jnp.sum(z * w3_ref[...], axis=1) + b3_ref[0, 0]
    o_ref[...] = jax.nn.sigmoid(z).reshape(_MLP_BLOCK // 128, 128)


def _mlp(h, W1, b1, W2, b2, W3, b3):
    B, D = h.shape
    grid = (B // _MLP_BLOCK,)
    out2d = pl.pallas_call(
        _mlp_body,
        grid=grid,
        in_specs=[
            pl.BlockSpec((_MLP_BLOCK, D), lambda i: (i, 0)),
            pl.BlockSpec(W1.shape, lambda i: (0, 0)),
            pl.BlockSpec((1, b1.shape[0]), lambda i: (0, 0)),
            pl.BlockSpec(W2.shape, lambda i: (0, 0)),
            pl.BlockSpec((1, b2.shape[0]), lambda i: (0, 0)),
            pl.BlockSpec((1, W3.shape[0]), lambda i: (0, 0)),
            pl.BlockSpec((1, b3.shape[0]), lambda i: (0, 0)),
        ],
        out_specs=pl.BlockSpec((_MLP_BLOCK // 128, 128), lambda i: (i, 0)),
        out_shape=jax.ShapeDtypeStruct((B // 128, 128), jnp.float32),
    )(h, W1, b1.reshape(1, -1), W2, b2.reshape(1, -1), W3.reshape(1, -1),
      b3.reshape(1, -1))
    return out2d.reshape(B, 1)


def kernel(x, table, W1, b1, W2, b2, W3, b3):
    x = x.astype(jnp.int32)
    h = _sc_gather(table, x)
    return _mlp(h, W1, b1, W2, b2, W3, b3)


# speedup vs baseline: 1.5174x; 1.0060x over previous
import functools

import jax
import jax.numpy as jnp
from jax.experimental import pallas as pl
from jax.experimental.pallas import tpu as pltpu
from jax.experimental.pallas import tpu_sc as plsc

_GATHER_WINDOW = 128
_MLP_BLOCK = 8192


_NUM_UNITS = 32


def _sc_gather(table, idx):
    B = idx.shape[0]
    D = table.shape[1]
    per_unit = B // _NUM_UNITS
    n_win = per_unit // _GATHER_WINDOW
    mesh = plsc.VectorSubcoreMesh(core_axis_name="c", subcore_axis_name="s")

    row_scratch = [
        pltpu.VMEM((_GATHER_WINDOW, D), table.dtype) for _ in range(n_win)
    ]

    @functools.partial(
        pl.kernel,
        out_type=jax.ShapeDtypeStruct((B, D), table.dtype),
        mesh=mesh,
        scratch_types=[pltpu.VMEM((per_unit,), jnp.int32)]
        + row_scratch
        + [pltpu.SemaphoreType.DMA, pltpu.SemaphoreType.DMA],
    )
    def gather_kernel(table_hbm, i_hbm, o_hbm, idx_v, *rest):
        rows = rest[:n_win]
        sem_g, sem_o = rest[n_win], rest[n_win + 1]
        wid = jax.lax.axis_index("s") * 2 + jax.lax.axis_index("c")
        base = wid * per_unit
        pltpu.sync_copy(i_hbm.at[pl.ds(base, per_unit)], idx_v)
        gathers = [
            pltpu.async_copy(
                table_hbm.at[idx_v.at[pl.ds(k * _GATHER_WINDOW, _GATHER_WINDOW)]],
                rows[k],
                sem_g,
            )
            for k in range(n_win)
        ]
        writes = []
        for k in range(n_win):
            gathers[k].wait()
            writes.append(
                pltpu.async_copy(
                    rows[k],
                    o_hbm.at[pl.ds(base + k * _GATHER_WINDOW, _GATHER_WINDOW)],
                    sem_o,
                )
            )
        for w in writes:
            w.wait()

    return gather_kernel(table, idx)


def _mlp_body(h_ref, w1_ref, b1_ref, w2_ref, b2_ref, w3_ref, b3_ref, o_ref):
    h = h_ref[...]
    z = jnp.dot(h, w1_ref[...], preferred_element_type=jnp.float32) + b1_ref[...]
    z = jnp.maximum(z, 0.0)
    z = jnp.dot(z, w2_ref[...], preferred_element_type=jnp.float32) + b2_ref[...]
    z = jnp.maximum(z, 0.0)
    z = jnp.sum(z * w3_ref[...], axis=1) + b3_ref[0, 0]
    o_ref[...] = jax.nn.sigmoid(z).reshape(_MLP_BLOCK // 128, 128)


def _mlp(h, W1, b1, W2, b2, W3, b3):
    B, D = h.shape
    grid = (B // _MLP_BLOCK,)
    out2d = pl.pallas_call(
        _mlp_body,
        grid=grid,
        in_specs=[
            pl.BlockSpec((_MLP_BLOCK, D), lambda i: (i, 0)),
            pl.BlockSpec(W1.shape, lambda i: (0, 0)),
            pl.BlockSpec((1, b1.shape[0]), lambda i: (0, 0)),
            pl.BlockSpec(W2.shape, lambda i: (0, 0)),
            pl.BlockSpec((1, b2.shape[0]), lambda i: (0, 0)),
            pl.BlockSpec((1, W3.shape[0]), lambda i: (0, 0)),
            pl.BlockSpec((1, b3.shape[0]), lambda i: (0, 0)),
        ],
        out_specs=pl.BlockSpec((_MLP_BLOCK // 128, 128), lambda i: (i, 0)),
        out_shape=jax.ShapeDtypeStruct((B // 128, 128), jnp.float32),
    )(h, W1, b1.reshape(1, -1), W2, b2.reshape(1, -1), W3.reshape(1, -1),
      b3.reshape(1, -1))
    return out2d.reshape(B, 1)


def kernel(x, table, W1, b1, W2, b2, W3, b3):
    x = x.astype(jnp.int32)
    h = _sc_gather(table, x)
    return _mlp(h, W1, b1, W2, b2, W3, b3)
